# Initial kernel scaffold; baseline (speedup 1.0000x reference)
import jax
import jax.numpy as jnp
from jax.experimental import pallas as pl


def kernel(x, edge_index, W1, b1, W2, b2, W3, b3):
    raise NotImplementedError("write your pallas kernel here")


# speedup vs baseline: 9.2827x; 9.2827x over previous
import functools

import jax
import jax.numpy as jnp
from jax import lax
from jax.experimental import pallas as pl
from jax.experimental.pallas import tpu as pltpu
from jax.experimental.pallas import tpu_sc as plsc

N = 10000
NP = 10240
E = 160000
NT = 16
NC = 2
ROWS_PER_TILE = NP // NT


def _mesh():
    return plsc.VectorSubcoreMesh(core_axis_name="c", subcore_axis_name="s")


def _make_agg_feat(n_fc, Fc, B, nb):
    fc_per_core = n_fc // NC

    @functools.partial(
        pl.kernel,
        mesh=_mesh(),
        out_type=jax.ShapeDtypeStruct((n_fc, NP, Fc), jnp.float32),
        scratch_types=[
            pltpu.VMEM((nb, B), jnp.int32),
            pltpu.VMEM((nb, 1, B), jnp.int32),
            pltpu.VMEM((B, Fc), jnp.float32),
            pltpu.VMEM_SHARED((NP, Fc), jnp.float32),
            pltpu.SemaphoreType.DMA,
        ],
    )
    def k(y_hbm, src_hbm, dst_hbm, zeros_hbm, z_hbm, src_v, dst_v, buf, acc, sem):
        c = lax.axis_index("c")
        s = lax.axis_index("s")
        pltpu.sync_copy(src_hbm.at[s], src_v)
        pltpu.sync_copy(dst_hbm.at[s], dst_v)
        row0 = s * ROWS_PER_TILE
        rows = pl.ds(row0, ROWS_PER_TILE)
        for fci in range(fc_per_core):
            fc = c * fc_per_core + fci
            pltpu.sync_copy(zeros_hbm.at[rows], acc.at[rows])
            plsc.subcore_barrier()

            def body(j, carry):
                pltpu.async_copy(y_hbm.at[fc].at[src_v.at[j]], buf, sem).wait()
                pltpu.sync_copy(buf, acc.at[dst_v.at[j, 0]], add=True)
                return carry

            lax.fori_loop(0, nb, body, 0)
            plsc.subcore_barrier()
            pltpu.sync_copy(acc.at[rows], z_hbm.at[fc].at[rows])

    return k


def _make_agg_edge(Fc, B, nb):
    @functools.partial(
        pl.kernel,
        mesh=_mesh(),
        out_type=jax.ShapeDtypeStruct((NC, NP, Fc), jnp.float32),
        scratch_types=[
            pltpu.VMEM((nb, B), jnp.int32),
            pltpu.VMEM((nb, 1, B), jnp.int32),
            pltpu.VMEM((B, Fc), jnp.float32),
            pltpu.VMEM_SHARED((NP, Fc), jnp.float32),
            pltpu.SemaphoreType.DMA,
        ],
    )
    def k(y_hbm, src_hbm, dst_hbm, zeros_hbm, z_hbm, src_v, dst_v, buf, acc, sem):
        c = lax.axis_index("c")
        s = lax.axis_index("s")
        pltpu.sync_copy(src_hbm.at[c].at[s], src_v)
        pltpu.sync_copy(dst_hbm.at[c].at[s], dst_v)
        row0 = s * ROWS_PER_TILE
        rows = pl.ds(row0, ROWS_PER_TILE)
        pltpu.sync_copy(zeros_hbm.at[rows], acc.at[rows])
        plsc.subcore_barrier()

        def body(j, carry):
            pltpu.async_copy(y_hbm.at[src_v.at[j]], buf, sem).wait()
            pltpu.sync_copy(buf, acc.at[dst_v.at[j, 0]], add=True)
            return carry

        lax.fori_loop(0, nb, body, 0)
        plsc.subcore_barrier()
        pltpu.sync_copy(acc.at[rows], z_hbm.at[c].at[rows])

    return k


def _make_deg(B, nb):
    Fc = 128

    @functools.partial(
        pl.kernel,
        mesh=_mesh(),
        out_type=jax.ShapeDtypeStruct((NC, NP, Fc), jnp.float32),
        scratch_types=[
            pltpu.VMEM((nb, 1, B), jnp.int32),
            pltpu.VMEM((B, Fc), jnp.float32),
            pltpu.VMEM_SHARED((NP, Fc), jnp.float32),
        ],
    )
    def k(dst_hbm, ones_hbm, zeros_hbm, z_hbm, dst_v, buf, acc):
        c = lax.axis_index("c")
        s = lax.axis_index("s")
        pltpu.sync_copy(dst_hbm.at[c].at[s], dst_v)
        pltpu.sync_copy(ones_hbm, buf)
        row0 = s * ROWS_PER_TILE
        rows = pl.ds(row0, ROWS_PER_TILE)
        pltpu.sync_copy(zeros_hbm.at[rows], acc.at[rows])
        plsc.subcore_barrier()

        def body(j, carry):
            pltpu.sync_copy(buf, acc.at[dst_v.at[j, 0]], add=True)
            return carry

        lax.fori_loop(0, nb, body, 0)
        plsc.subcore_barrier()
        pltpu.sync_copy(acc.at[rows], z_hbm.at[c].at[rows])

    return k


BM = 1024


def _dinv_kernel(degp_ref, o_ref):
    v = degp_ref[...]
    deg = v[0, :, 0:1] + v[1, :, 0:1] + 1.0
    o_ref[...] = lax.rsqrt(deg)


def _dinv(degp):
    return pl.pallas_call(
        _dinv_kernel,
        grid=(NP // BM,),
        in_specs=[pl.BlockSpec((NC, BM, 128), lambda m: (0, m, 0))],
        out_specs=pl.BlockSpec((BM, 1), lambda m: (m, 0)),
        out_shape=jax.ShapeDtypeStruct((NP, 1), jnp.float32),
    )(degp)


def _scale_chunk_kernel(x_ref, d_ref, o_ref):
    o_ref[...] = (x_ref[...] * d_ref[...])[None]


def _scale_chunk(x, dinv, n_fc):
    return pl.pallas_call(
        _scale_chunk_kernel,
        grid=(NP // BM, n_fc),
        in_specs=[
            pl.BlockSpec((BM, 128), lambda m, f: (m, f)),
            pl.BlockSpec((BM, 1), lambda m, f: (m, 0)),
        ],
        out_specs=pl.BlockSpec((1, BM, 128), lambda m, f: (f, m, 0)),
        out_shape=jax.ShapeDtypeStruct((n_fc, NP, 128), jnp.float32),
    )(x, dinv)


def _combine_kernel(z_ref, y_ref, d_ref, o_ref):
    o_ref[...] = d_ref[...] * (z_ref[0] + y_ref[0])


def _combine(z, y, dinv, F):
    n_fc = F // 128
    return pl.pallas_call(
        _combine_kernel,
        grid=(NP // BM, n_fc),
        in_specs=[
            pl.BlockSpec((1, BM, 128), lambda m, f: (f, m, 0)),
            pl.BlockSpec((1, BM, 128), lambda m, f: (f, m, 0)),
            pl.BlockSpec((BM, 1), lambda m, f: (m, 0)),
        ],
        out_specs=pl.BlockSpec((BM, 128), lambda m, f: (m, f)),
        out_shape=jax.ShapeDtypeStruct((NP, F), jnp.float32),
    )(z, y, dinv)


def _mm_bias_relu_kernel(a_ref, w_ref, b_ref, o_ref):
    acc = jnp.dot(a_ref[...], w_ref[...], preferred_element_type=jnp.float32)
    o_ref[...] = jnp.maximum(acc + b_ref[...], 0.0)


def _mm_bias_relu(a, w, b):
    K = a.shape[1]
    F = w.shape[1]
    return pl.pallas_call(
        _mm_bias_relu_kernel,
        grid=(NP // BM, F // 128),
        in_specs=[
            pl.BlockSpec((BM, K), lambda m, f: (m, 0)),
            pl.BlockSpec((K, 128), lambda m, f: (0, f)),
            pl.BlockSpec((1, 128), lambda m, f: (0, f)),
        ],
        out_specs=pl.BlockSpec((BM, 128), lambda m, f: (m, f)),
        out_shape=jax.ShapeDtypeStruct((NP, F), jnp.float32),
    )(a, w, b)


def _mm_scale_chunk_kernel(a_ref, w_ref, d_ref, o_ref):
    acc = jnp.dot(a_ref[...], w_ref[...], preferred_element_type=jnp.float32)
    o_ref[...] = (d_ref[...] * acc)[None]


def _mm_scale_chunk(a, w, dinv):
    K = a.shape[1]
    F = w.shape[1]
    n_fc = F // 128
    return pl.pallas_call(
        _mm_scale_chunk_kernel,
        grid=(NP // BM, n_fc),
        in_specs=[
            pl.BlockSpec((BM, K), lambda m, f: (m, 0)),
            pl.BlockSpec((K, 128), lambda m, f: (0, f)),
            pl.BlockSpec((BM, 1), lambda m, f: (m, 0)),
        ],
        out_specs=pl.BlockSpec((1, BM, 128), lambda m, f: (f, m, 0)),
        out_shape=jax.ShapeDtypeStruct((n_fc, NP, 128), jnp.float32),
    )(a, w, dinv)


def _combine3_kernel(z_ref, y_ref, d_ref, b_ref, o_ref):
    o_ref[...] = d_ref[...] * (z_ref[0] + z_ref[1] + y_ref[...]) + b_ref[...]


def _combine3(z, y, dinv, b128):
    return pl.pallas_call(
        _combine3_kernel,
        grid=(NP // BM,),
        in_specs=[
            pl.BlockSpec((NC, BM, 128), lambda m: (0, m, 0)),
            pl.BlockSpec((BM, 128), lambda m: (m, 0)),
            pl.BlockSpec((BM, 1), lambda m: (m, 0)),
            pl.BlockSpec((1, 128), lambda m: (0, 0)),
        ],
        out_specs=pl.BlockSpec((BM, 128), lambda m: (m, 0)),
        out_shape=jax.ShapeDtypeStruct((NP, 128), jnp.float32),
    )(z, y, dinv, b128)


def _combine_bias_relu_kernel(z_ref, y_ref, d_ref, b_ref, o_ref):
    o_ref[...] = jnp.maximum(
        d_ref[...] * (z_ref[0] + y_ref[0]) + b_ref[...], 0.0
    )


def _combine_bias_relu(z, y, dinv, b, F):
    n_fc = F // 128
    return pl.pallas_call(
        _combine_bias_relu_kernel,
        grid=(NP // BM, n_fc),
        in_specs=[
            pl.BlockSpec((1, BM, 128), lambda m, f: (f, m, 0)),
            pl.BlockSpec((1, BM, 128), lambda m, f: (f, m, 0)),
            pl.BlockSpec((BM, 1), lambda m, f: (m, 0)),
            pl.BlockSpec((1, 128), lambda m, f: (0, f)),
        ],
        out_specs=pl.BlockSpec((BM, 128), lambda m, f: (m, f)),
        out_shape=jax.ShapeDtypeStruct((NP, F), jnp.float32),
    )(z, y, dinv, b)


B_FEAT = 80
NB_FEAT = E // (NT * B_FEAT)
B_EDGE = 40
NB_EDGE = E // (NC * NT * B_EDGE)

_agg1 = _make_agg_feat(2, 128, B_FEAT, NB_FEAT)
_agg2 = _make_agg_feat(4, 128, B_FEAT, NB_FEAT)
_agg3 = _make_agg_edge(128, B_EDGE, NB_EDGE)
_deg = _make_deg(B_EDGE, NB_EDGE)


@jax.jit
def kernel(x, edge_index, W1, b1, W2, b2, W3, b3):
    ei = edge_index.astype(jnp.int32)
    src = ei[0]
    dst = ei[1]
    src_f = src.reshape(NT, NB_FEAT, B_FEAT)
    dst_f = dst.reshape(NT, NB_FEAT, 1, B_FEAT)
    src_e = src.reshape(NC, NT, NB_EDGE, B_EDGE)
    dst_e = dst.reshape(NC, NT, NB_EDGE, 1, B_EDGE)

    z128 = jnp.zeros((NP, 128), jnp.float32)
    ones128 = jnp.ones((B_EDGE, 128), jnp.float32)
    xp = jnp.zeros((NP, 256), jnp.float32).at[:N].set(x)

    degp = _deg(dst_e, ones128, z128)
    dinv = _dinv(degp)

    y1 = _scale_chunk(xp, dinv, 2)
    zz1 = _agg1(y1, src_f, dst_f, z128)
    g1 = _combine(zz1, y1, dinv, 256)
    h1 = _mm_bias_relu(g1, W1, b1.reshape(1, 512))

    y2 = _mm_scale_chunk(h1, W2, dinv)
    zz2 = _agg2(y2, src_f, dst_f, z128)
    h2 = _combine_bias_relu(zz2, y2, dinv, b2.reshape(1, 512), 512)

    W3p = jnp.zeros((512, 128), jnp.float32).at[:, :2].set(W3)
    y3full = _mm_scale_chunk(h2, W3p, dinv)
    y3 = y3full[0]
    zz3 = _agg3(y3, src_e, dst_e, z128)
    b128 = jnp.zeros((1, 128), jnp.float32).at[0, :2].set(b3)
    out = _combine3(zz3, y3, dinv, b128)
    return out[:N, :2]
